# Initial kernel scaffold; baseline (speedup 1.0000x reference)
import jax
import jax.numpy as jnp
from jax.experimental import pallas as pl


def kernel(x, tables):
    raise NotImplementedError("write your pallas kernel here")


# speedup vs baseline: 15.0352x; 15.0352x over previous
import functools

import jax
import jax.numpy as jnp
from jax import lax
from jax.experimental import pallas as pl
from jax.experimental.pallas import tpu as pltpu
from jax.experimental.pallas import tpu_sc as plsc

INPUT_DIM = 3
NUM_LEVELS = 16
LEVEL_DIM = 2
BASE_RES = 16
MAX_RES = 2048
LOG2_HASH = 19
HASH_SIZE = 2 ** LOG2_HASH
_b = (MAX_RES / BASE_RES) ** (1.0 / (NUM_LEVELS - 1))
_RESOLUTIONS = [float(int(BASE_RES * _b ** i)) for i in range(NUM_LEVELS)]
_P1 = -1640531535
_P2 = 805459861
_MASK = HASH_SIZE - 1

_RES_DUP = [_RESOLUTIONS[k // 2] for k in range(2 * NUM_LEVELS)]
_OFF_DUP = [((k // 2) << (LOG2_HASH + 1)) + (k & 1) for k in range(2 * NUM_LEVELS)]


def _hash_gather_kernel(B, C, x_hbm, tab_hbm, resd_hbm, offd_hbm, out_hbm,
                        xv, idx, rows, rdv, odv, sem):
    NW = 32
    PW = B // NW
    nchunk = PW // C
    CN = C * NUM_LEVELS * LEVEL_DIM

    cid = lax.axis_index("c")
    sid = lax.axis_index("s")
    wid = sid * 2 + cid
    pltpu.sync_copy(resd_hbm, rdv)
    pltpu.sync_copy(offd_hbm, odv)
    res = [rdv[pl.ds(16 * k, 16)] for k in range(2)]
    off = [odv[pl.ds(16 * k, 16)] for k in range(2)]

    def chunk_body(c, _):
        base_p = wid * PW + c * C
        pltpu.sync_copy(x_hbm.at[pl.ds(base_p * INPUT_DIM, C * INPUT_DIM)], xv)

        def grp_body(g, _):
            v = [xv[pl.ds(g * 48 + 16 * k, 16)] for k in range(3)]
            for j in range(16):
                x0 = v[(3 * j) // 16][(3 * j) % 16]
                x1 = v[(3 * j + 1) // 16][(3 * j + 1) % 16]
                x2 = v[(3 * j + 2) // 16][(3 * j + 2) % 16]
                for k in range(2):
                    f0 = (x0 * res[k]).astype(jnp.int32)
                    f1 = (x1 * res[k]).astype(jnp.int32)
                    f2 = (x2 * res[k]).astype(jnp.int32)
                    h = (f0 + f1 * _P1 + f2 * _P2) & _MASK
                    e = (h << 1) + off[k]
                    idx[pl.ds((g * 16 + j) * 32 + 16 * k, 16)] = e
            return 0

        lax.fori_loop(jnp.int32(0), jnp.int32(C // 16), grp_body, 0)
        pltpu.async_copy(tab_hbm.at[idx], rows, sem).wait()
        pltpu.sync_copy(rows, out_hbm.at[pl.ds(base_p * 32, CN)])
        return 0

    lax.fori_loop(jnp.int32(0), jnp.int32(nchunk), chunk_body, 0)


def kernel(x, tables):
    B = x.shape[0]
    C = 1024
    tab = tables.reshape(NUM_LEVELS * HASH_SIZE * LEVEL_DIM)
    CN = C * NUM_LEVELS * LEVEL_DIM

    mesh = plsc.VectorSubcoreMesh(core_axis_name="c", subcore_axis_name="s")
    out = pl.kernel(
        functools.partial(_hash_gather_kernel, B, C),
        out_type=jax.ShapeDtypeStruct((B * NUM_LEVELS * LEVEL_DIM,),
                                      jnp.float32),
        mesh=mesh,
        scratch_types=[
            pltpu.VMEM((C * INPUT_DIM,), jnp.float32),
            pltpu.VMEM((CN,), jnp.int32),
            pltpu.VMEM((CN,), jnp.float32),
            pltpu.VMEM((2 * NUM_LEVELS,), jnp.float32),
            pltpu.VMEM((2 * NUM_LEVELS,), jnp.int32),
            pltpu.SemaphoreType.DMA,
        ],
    )(x.reshape(B * INPUT_DIM), tab,
      jnp.array(_RES_DUP, dtype=jnp.float32),
      jnp.array(_OFF_DUP, dtype=jnp.int32))
    return out.reshape(B, NUM_LEVELS * LEVEL_DIM)


# speedup vs baseline: 59.7853x; 3.9764x over previous
import functools

import jax
import jax.numpy as jnp
from jax import lax
from jax.experimental import pallas as pl
from jax.experimental.pallas import tpu as pltpu
from jax.experimental.pallas import tpu_sc as plsc

INPUT_DIM = 3
NUM_LEVELS = 16
LEVEL_DIM = 2
BASE_RES = 16
MAX_RES = 2048
LOG2_HASH = 19
HASH_SIZE = 2 ** LOG2_HASH
_b = (MAX_RES / BASE_RES) ** (1.0 / (NUM_LEVELS - 1))
_RESOLUTIONS = [float(int(BASE_RES * _b ** i)) for i in range(NUM_LEVELS)]
_P1 = -1640531535
_P2 = 805459861
_MASK = HASH_SIZE - 1

_RES_DUP = [_RESOLUTIONS[k // 2] for k in range(2 * NUM_LEVELS)]
_OFF_DUP = [((k // 2) << (LOG2_HASH + 1)) + ((k & 1) << 7)
            for k in range(2 * NUM_LEVELS)]


def _hash_gather_kernel(B, C, x_hbm, tab_hbm, resd_hbm, offd_hbm, out_hbm,
                        xv, idx, rows, rdv, odv, sem):
    NW = 32
    PW = B // NW
    nchunk = PW // C
    CN = C * NUM_LEVELS * LEVEL_DIM

    cid = lax.axis_index("c")
    sid = lax.axis_index("s")
    wid = sid * 2 + cid
    pltpu.sync_copy(resd_hbm, rdv)
    pltpu.sync_copy(offd_hbm, odv)
    res = [rdv[pl.ds(16 * k, 16)] for k in range(2)]
    off = [odv[pl.ds(16 * k, 16)] for k in range(2)]

    def chunk_body(c, _):
        base_p = wid * PW + c * C
        pltpu.sync_copy(x_hbm.at[pl.ds(base_p * INPUT_DIM, C * INPUT_DIM)], xv)

        def grp_body(g, _):
            v = [xv[pl.ds(g * 48 + 16 * k, 16)] for k in range(3)]
            for j in range(16):
                x0 = v[(3 * j) // 16][(3 * j) % 16]
                x1 = v[(3 * j + 1) // 16][(3 * j + 1) % 16]
                x2 = v[(3 * j + 2) // 16][(3 * j + 2) % 16]
                for k in range(2):
                    f0 = (x0 * res[k]).astype(jnp.int32)
                    f1 = (x1 * res[k]).astype(jnp.int32)
                    f2 = (x2 * res[k]).astype(jnp.int32)
                    h = (f0 + f1 * _P1 + f2 * _P2) & _MASK
                    lo = h & 127
                    e = (((h - lo) << 1) + lo) + off[k]
                    idx[pl.ds((g * 16 + j) * 32 + 16 * k, 16)] = e
            return 0

        lax.fori_loop(jnp.int32(0), jnp.int32(C // 16), grp_body, 0)
        pltpu.async_copy(tab_hbm.at[idx], rows, sem).wait()
        pltpu.sync_copy(rows, out_hbm.at[pl.ds(base_p * 32, CN)])
        return 0

    lax.fori_loop(jnp.int32(0), jnp.int32(nchunk), chunk_body, 0)


def kernel(x, tables):
    B = x.shape[0]
    C = 1024
    tab = (tables.reshape(NUM_LEVELS, HASH_SIZE // 128, 128, LEVEL_DIM)
           .transpose(0, 1, 3, 2)
           .reshape(NUM_LEVELS * HASH_SIZE * LEVEL_DIM))
    CN = C * NUM_LEVELS * LEVEL_DIM

    mesh = plsc.VectorSubcoreMesh(core_axis_name="c", subcore_axis_name="s")
    out = pl.kernel(
        functools.partial(_hash_gather_kernel, B, C),
        out_type=jax.ShapeDtypeStruct((B * NUM_LEVELS * LEVEL_DIM,),
                                      jnp.float32),
        mesh=mesh,
        scratch_types=[
            pltpu.VMEM((C * INPUT_DIM,), jnp.float32),
            pltpu.VMEM((CN,), jnp.int32),
            pltpu.VMEM((CN,), jnp.float32),
            pltpu.VMEM((2 * NUM_LEVELS,), jnp.float32),
            pltpu.VMEM((2 * NUM_LEVELS,), jnp.int32),
            pltpu.SemaphoreType.DMA,
        ],
    )(x.reshape(B * INPUT_DIM), tab,
      jnp.array(_RES_DUP, dtype=jnp.float32),
      jnp.array(_OFF_DUP, dtype=jnp.int32))
    return out.reshape(B, NUM_LEVELS * LEVEL_DIM)


# speedup vs baseline: 60.7820x; 1.0167x over previous
import functools

import jax
import jax.numpy as jnp
from jax import lax
from jax.experimental import pallas as pl
from jax.experimental.pallas import tpu as pltpu
from jax.experimental.pallas import tpu_sc as plsc

INPUT_DIM = 3
NUM_LEVELS = 16
LEVEL_DIM = 2
BASE_RES = 16
MAX_RES = 2048
LOG2_HASH = 19
HASH_SIZE = 2 ** LOG2_HASH
_b = (MAX_RES / BASE_RES) ** (1.0 / (NUM_LEVELS - 1))
_RESOLUTIONS = [float(int(BASE_RES * _b ** i)) for i in range(NUM_LEVELS)]
_P1 = -1640531535
_P2 = 805459861
_MASK = HASH_SIZE - 1

_RES_DUP = [_RESOLUTIONS[k // 2] for k in range(2 * NUM_LEVELS)]
_OFF_DUP = [((k // 2) << (LOG2_HASH + 1)) + ((k & 1) << 7)
            for k in range(2 * NUM_LEVELS)]


def _hash_gather_kernel(B, C, x_hbm, tab_hbm, resd_hbm, offd_hbm, out_hbm,
                        xv, idx, rows, rpad, rdv, odv, sem):
    NW = 32
    PW = B // NW
    nchunk = PW // C
    CN = C * NUM_LEVELS * LEVEL_DIM

    cid = lax.axis_index("c")
    sid = lax.axis_index("s")
    wid = sid * 2 + cid
    pltpu.sync_copy(resd_hbm, rdv)
    pltpu.sync_copy(offd_hbm, odv)
    res = [rdv[pl.ds(16 * k, 16)] for k in range(2)]
    off = [odv[pl.ds(16 * k, 16)] for k in range(2)]

    def chunk_body(c, _):
        base_p = wid * PW + c * C
        pltpu.sync_copy(x_hbm.at[pl.ds(base_p * INPUT_DIM, C * INPUT_DIM)], xv)

        def grp_body(g, _):
            v = [xv[pl.ds(g * 48 + 16 * k, 16)] for k in range(3)]
            for j in range(16):
                x0 = v[(3 * j) // 16][(3 * j) % 16]
                x1 = v[(3 * j + 1) // 16][(3 * j + 1) % 16]
                x2 = v[(3 * j + 2) // 16][(3 * j + 2) % 16]
                for k in range(2):
                    f0 = (x0 * res[k]).astype(jnp.int32)
                    f1 = (x1 * res[k]).astype(jnp.int32)
                    f2 = (x2 * res[k]).astype(jnp.int32)
                    h = (f0 + f1 * _P1 + f2 * _P2) & _MASK
                    lo = h & 127
                    e = (((h - lo) << 1) + lo) + off[k]
                    idx[pl.ds((g * 16 + j) * 32 + 16 * k, 16)] = e
            return 0

        lax.fori_loop(jnp.int32(0), jnp.int32(C // 16), grp_body, 0)
        pltpu.async_copy(tab_hbm.at[idx], rows, sem).wait()

        def pk_body(p, _):
            a = rows[pl.ds(p * 32, 16)]
            b = rows[pl.ds(p * 32 + 16, 16)]
            rpad[p, pl.ds(0, 16)] = a
            rpad[p, pl.ds(16, 16)] = b
            return 0

        lax.fori_loop(jnp.int32(0), jnp.int32(C), pk_body, 0)
        pltpu.sync_copy(rpad, out_hbm.at[pl.ds(base_p, C), :])
        return 0

    lax.fori_loop(jnp.int32(0), jnp.int32(nchunk), chunk_body, 0)


def kernel(x, tables):
    B = x.shape[0]
    C = 512
    tab = (tables.reshape(NUM_LEVELS, HASH_SIZE // 128, 128, LEVEL_DIM)
           .transpose(0, 1, 3, 2)
           .reshape(NUM_LEVELS * HASH_SIZE * LEVEL_DIM))
    CN = C * NUM_LEVELS * LEVEL_DIM

    mesh = plsc.VectorSubcoreMesh(core_axis_name="c", subcore_axis_name="s")
    out = pl.kernel(
        functools.partial(_hash_gather_kernel, B, C),
        out_type=jax.ShapeDtypeStruct((B, 128), jnp.float32),
        mesh=mesh,
        scratch_types=[
            pltpu.VMEM((C * INPUT_DIM,), jnp.float32),
            pltpu.VMEM((CN,), jnp.int32),
            pltpu.VMEM((CN,), jnp.float32),
            pltpu.VMEM((C, 128), jnp.float32),
            pltpu.VMEM((2 * NUM_LEVELS,), jnp.float32),
            pltpu.VMEM((2 * NUM_LEVELS,), jnp.int32),
            pltpu.SemaphoreType.DMA,
        ],
    )(x.reshape(B * INPUT_DIM), tab,
      jnp.array(_RES_DUP, dtype=jnp.float32),
      jnp.array(_OFF_DUP, dtype=jnp.int32))
    return out[:, :NUM_LEVELS * LEVEL_DIM]


# speedup vs baseline: 142.1487x; 2.3387x over previous
import functools

import jax
import jax.numpy as jnp
from jax import lax
from jax.experimental import pallas as pl
from jax.experimental.pallas import tpu as pltpu
from jax.experimental.pallas import tpu_sc as plsc

INPUT_DIM = 3
NUM_LEVELS = 16
LEVEL_DIM = 2
BASE_RES = 16
MAX_RES = 2048
LOG2_HASH = 19
HASH_SIZE = 2 ** LOG2_HASH
_b = (MAX_RES / BASE_RES) ** (1.0 / (NUM_LEVELS - 1))
_RESOLUTIONS = [float(int(BASE_RES * _b ** i)) for i in range(NUM_LEVELS)]
_P1 = -1640531535
_P2 = 805459861
_MASK = HASH_SIZE - 1
_NC = NUM_LEVELS * LEVEL_DIM


def _hash_gather_kernel(B, C, xt_hbm, tab_hbm, out_hbm, xv, idx, rows, sem):
    NW = 32
    PW = B // NW
    nchunk = PW // C

    cid = lax.axis_index("c")
    sid = lax.axis_index("s")
    wid = sid * 2 + cid

    def chunk_body(c, _):
        base_p = wid * PW + c * C
        pltpu.sync_copy(xt_hbm.at[:, pl.ds(base_p, C)], xv)

        def grp_body(g, _):
            xa = xv[0, pl.ds(g * 16, 16)]
            xb = xv[1, pl.ds(g * 16, 16)]
            xc = xv[2, pl.ds(g * 16, 16)]
            goff = (g >> 3) * 1024 + (g & 7) * 16
            for l in range(NUM_LEVELS):
                r = _RESOLUTIONS[l]
                f0 = (xa * r).astype(jnp.int32)
                f1 = (xb * r).astype(jnp.int32)
                f2 = (xc * r).astype(jnp.int32)
                h = (f0 + f1 * _P1 + f2 * _P2) & _MASK
                lo = h & 127
                e0 = ((h - lo) << 1) + lo + (l << (LOG2_HASH + 1))
                for d in range(LEVEL_DIM):
                    cc = 2 * l + d
                    coff = (cc >> 3) * (C * 8) + (cc & 7) * 128
                    idx[pl.ds(goff + coff, 16)] = e0 + d * 128
            return 0

        lax.fori_loop(jnp.int32(0), jnp.int32(C // 16), grp_body, 0)
        pltpu.async_copy(tab_hbm.at[idx], rows, sem).wait()
        for cb in range(4):
            pltpu.sync_copy(
                rows.at[pl.ds(cb * (C * 8), C * 8)],
                out_hbm.at[pl.ds(cb * (B * 8) + (base_p >> 7) * 1024, C * 8)])
        return 0

    lax.fori_loop(jnp.int32(0), jnp.int32(nchunk), chunk_body, 0)


def kernel(x, tables):
    B = x.shape[0]
    C = 1024
    tab = (tables.reshape(NUM_LEVELS, HASH_SIZE // 128, 128, LEVEL_DIM)
           .transpose(0, 1, 3, 2)
           .reshape(NUM_LEVELS * HASH_SIZE * LEVEL_DIM))
    CN = C * _NC

    mesh = plsc.VectorSubcoreMesh(core_axis_name="c", subcore_axis_name="s")
    out = pl.kernel(
        functools.partial(_hash_gather_kernel, B, C),
        out_type=jax.ShapeDtypeStruct((B * _NC,), jnp.float32),
        mesh=mesh,
        scratch_types=[
            pltpu.VMEM((INPUT_DIM, C), jnp.float32),
            pltpu.VMEM((CN,), jnp.int32),
            pltpu.VMEM((CN,), jnp.float32),
            pltpu.SemaphoreType.DMA,
        ],
    )(x.T, tab)
    return (out.reshape(4, B // 128, 8, 128)
            .transpose(1, 3, 0, 2)
            .reshape(B, _NC))


# speedup vs baseline: 148.2252x; 1.0427x over previous
import functools

import jax
import jax.numpy as jnp
from jax import lax
from jax.experimental import pallas as pl
from jax.experimental.pallas import tpu as pltpu
from jax.experimental.pallas import tpu_sc as plsc

INPUT_DIM = 3
NUM_LEVELS = 16
LEVEL_DIM = 2
BASE_RES = 16
MAX_RES = 2048
LOG2_HASH = 19
HASH_SIZE = 2 ** LOG2_HASH
_b = (MAX_RES / BASE_RES) ** (1.0 / (NUM_LEVELS - 1))
_RESOLUTIONS = [float(int(BASE_RES * _b ** i)) for i in range(NUM_LEVELS)]
_P1 = -1640531535
_P2 = 805459861
_MASK = HASH_SIZE - 1
_NC = NUM_LEVELS * LEVEL_DIM


def _hash_gather_kernel(B, C, xt_hbm, tab_hbm, out_hbm,
                        xv, i00, i01, i02, i03, i10, i11, i12, i13,
                        r00, r01, r02, r03, r10, r11, r12, r13,
                        gsem0, gsem1, osem0, osem1):
    NW = 32
    PW = B // NW
    nchunk = PW // C
    PL = C * 8

    cid = lax.axis_index("c")
    sid = lax.axis_index("s")
    wid = sid * 2 + cid
    idx_b = ((i00, i01, i02, i03), (i10, i11, i12, i13))
    rows_b = ((r00, r01, r02, r03), (r10, r11, r12, r13))
    gsem_b = (gsem0, gsem1)
    osem_b = (osem0, osem1)

    def compute_idx(c, idx):
        base_p = wid * PW + c * C
        pltpu.sync_copy(xt_hbm.at[:, pl.ds(base_p, C)], xv)

        def grp_body(g, _):
            xa = xv[0, pl.ds(g * 16, 16)]
            xb = xv[1, pl.ds(g * 16, 16)]
            xc = xv[2, pl.ds(g * 16, 16)]
            goff = (g >> 3) * 1024 + (g & 7) * 16
            for l in range(NUM_LEVELS):
                r = _RESOLUTIONS[l]
                f0 = (xa * r).astype(jnp.int32)
                f1 = (xb * r).astype(jnp.int32)
                f2 = (xc * r).astype(jnp.int32)
                h = (f0 + f1 * _P1 + f2 * _P2) & _MASK
                lo = h & 127
                e0 = ((h - lo) << 1) + lo + (l << (LOG2_HASH + 1))
                for d in range(LEVEL_DIM):
                    cc = 2 * l + d
                    idx[cc >> 3][pl.ds(goff + (cc & 7) * 128, 16)] = e0 + d * 128
            return 0

        lax.fori_loop(jnp.int32(0), jnp.int32(C // 16), grp_body, 0)

    def start_gather(b):
        for j in range(4):
            pltpu.async_copy(tab_hbm.at[idx_b[b][j]], rows_b[b][j], gsem_b[b])

    def wait_gather(b):
        for j in range(4):
            pltpu.make_async_copy(out_hbm.at[jnp.int32(0), pl.ds(0, PL)],
                                  rows_b[b][j], gsem_b[b]).wait()

    def start_out(c, b):
        base_p = wid * PW + c * C
        for j in range(4):
            pltpu.async_copy(
                rows_b[b][j],
                out_hbm.at[jnp.int32(j), pl.ds((base_p >> 7) * 1024, PL)],
                osem_b[b])

    def wait_out(b):
        for j in range(4):
            pltpu.make_async_copy(out_hbm.at[jnp.int32(0), pl.ds(0, PL)],
                                  rows_b[b][j], osem_b[b]).wait()

    compute_idx(jnp.int32(0), idx_b[0])
    start_gather(0)
    compute_idx(jnp.int32(1), idx_b[1])
    wait_gather(0)
    start_gather(1)
    start_out(jnp.int32(0), 0)

    def pair_body(p, _):
        for b in range(2):
            c = 2 * p + b
            wait_out(b)
            compute_idx(c, idx_b[b])
            wait_gather(1 - b)
            start_gather(b)
            start_out(c - 1, 1 - b)
        return 0

    lax.fori_loop(jnp.int32(1), jnp.int32(nchunk // 2), pair_body, 0)

    wait_gather(1)
    start_out(jnp.int32(nchunk - 1), 1)
    wait_out(0)
    wait_out(1)


def kernel(x, tables):
    B = x.shape[0]
    C = 512
    tab = (tables.reshape(NUM_LEVELS, HASH_SIZE // 128, 128, LEVEL_DIM)
           .transpose(0, 1, 3, 2)
           .reshape(NUM_LEVELS * HASH_SIZE * LEVEL_DIM))
    PL = C * 8

    mesh = plsc.VectorSubcoreMesh(core_axis_name="c", subcore_axis_name="s")
    out = pl.kernel(
        functools.partial(_hash_gather_kernel, B, C),
        out_type=jax.ShapeDtypeStruct((4, B * 8), jnp.float32),
        mesh=mesh,
        scratch_types=(
            [pltpu.VMEM((INPUT_DIM, C), jnp.float32)]
            + [pltpu.VMEM((PL,), jnp.int32) for _ in range(8)]
            + [pltpu.VMEM((PL,), jnp.float32) for _ in range(8)]
            + [pltpu.SemaphoreType.DMA for _ in range(4)]
        ),
    )(x.T, tab)
    return (out.reshape(4, B // 128, 8, 128)
            .transpose(1, 3, 0, 2)
            .reshape(B, _NC))
